# baseline (speedup 1.0000x reference)
import functools

import jax
import jax.numpy as jnp
from jax import lax
from jax.experimental import pallas as pl
from jax.experimental.pallas import tpu as pltpu
from jax.experimental.pallas import tpu_sc as plsc

D = 64
B = 16384
PACK = 128
NC = 2
NS = 16
L = 16
NW = NC * NS
BPW = B // NW
HALF = BPW // 2
CHUNK = 128
NCHUNK = HALF // CHUNK

_mesh = plsc.VectorSubcoreMesh(core_axis_name="c", subcore_axis_name="s")


@functools.partial(
    pl.kernel,
    out_type=jax.ShapeDtypeStruct((B,), jnp.float32),
    mesh=_mesh,
    compiler_params=pltpu.CompilerParams(needs_layout_passes=False),
    scratch_types=[
        pltpu.VMEM((NCHUNK, CHUNK), jnp.int32),
        pltpu.VMEM((NCHUNK, CHUNK), jnp.int32),
        pltpu.VMEM((NCHUNK, CHUNK), jnp.int32),
        pltpu.VMEM((NCHUNK, CHUNK), jnp.int32),
        pltpu.VMEM((HALF, PACK), jnp.float32),
        pltpu.VMEM((HALF, PACK), jnp.float32),
        pltpu.VMEM((BPW,), jnp.float32),
        pltpu.SemaphoreType.DMA,
        pltpu.SemaphoreType.DMA,
    ],
)
def _sc_dot_kernel(ti_hbm, cj_hbm, emb2_hbm, out_hbm,
                   pidx_t, pidx_c, ridx_t, ridx_c, trows, crows, outv,
                   sem_t, sem_c):
    wid = lax.axis_index("s") * NC + lax.axis_index("c")
    base = wid * BPW

    lanes = lax.iota(jnp.int32, L)

    def half_body(h, _):
        row0 = wid * (BPW // CHUNK) + h * NCHUNK
        pltpu.sync_copy(ti_hbm.at[pl.ds(row0, NCHUNK)], ridx_t)
        pltpu.sync_copy(cj_hbm.at[pl.ds(row0, NCHUNK)], ridx_c)

        def shift_body(v, _):
            j = v // (CHUNK // L)
            r = (v % (CHUNK // L)) * L
            pidx_t[j, pl.ds(r, L)] = lax.shift_right_logical(
                ridx_t[j, pl.ds(r, L)], 1)
            pidx_c[j, pl.ds(r, L)] = lax.shift_right_logical(
                ridx_c[j, pl.ds(r, L)], 1)
            return 0
        lax.fori_loop(0, NCHUNK * (CHUNK // L), shift_body, 0)

        copies = []
        for j in range(NCHUNK):
            copies.append(pltpu.async_copy(
                emb2_hbm.at[pidx_t.at[j]],
                trows.at[pl.ds(j * CHUNK, CHUNK)], sem_t))
            copies.append(pltpu.async_copy(
                emb2_hbm.at[pidx_c.at[j]],
                crows.at[pl.ds(j * CHUNK, CHUNK)], sem_c))
        for cp in copies:
            cp.wait()

        def group_body(g, _):
            j = g // (CHUNK // L)
            r = (g % (CHUNK // L)) * L
            rows = g * L + lanes
            rawt = ridx_t[j, pl.ds(r, L)]
            rawc = ridx_c[j, pl.ds(r, L)]
            tbase = (rawt & 1) * D
            cbase = (rawc & 1) * D

            def dstep(d, acc):
                tv = plsc.load_gather(trows, [rows, tbase + d])
                cv = plsc.load_gather(crows, [rows, cbase + d])
                return acc + tv * cv

            acc = lax.fori_loop(0, D, dstep, jnp.zeros((L,), jnp.float32))
            outv[pl.ds(h * HALF + g * L, L)] = 1.0 / (1.0 + jnp.exp(-acc))
            return 0

        lax.fori_loop(0, HALF // L, group_body, 0)
        return 0

    lax.fori_loop(0, 2, half_body, 0)
    pltpu.sync_copy(outv, out_hbm.at[pl.ds(base, BPW)])


def kernel(target_i, context_j, label, emb):
    emb2 = emb.reshape(emb.shape[0] // 2, PACK)
    ti = target_i.reshape(B // CHUNK, CHUNK)
    cj = context_j.reshape(B // CHUNK, CHUNK)
    out = _sc_dot_kernel(ti, cj, emb2)
    return (out, label.astype(jnp.float32))


# speedup vs baseline: 1.6022x; 1.6022x over previous
import functools

import jax
import jax.numpy as jnp
from jax import lax
from jax.experimental import pallas as pl
from jax.experimental.pallas import tpu as pltpu
from jax.experimental.pallas import tpu_sc as plsc

D = 64
B = 16384
NC = 2
NS = 16
L = 16
NW = NC * NS
BPW = B // NW
NG = BPW // L
IDXCH = BPW // 128

_mesh = plsc.VectorSubcoreMesh(core_axis_name="c", subcore_axis_name="s")

_WIN_SCRATCH = [pltpu.VMEM((8, D), jnp.float32) for _ in range(2 * 2 * L)]


@functools.partial(
    pl.kernel,
    out_type=jax.ShapeDtypeStruct((B,), jnp.float32),
    mesh=_mesh,
    compiler_params=pltpu.CompilerParams(needs_layout_passes=False),
    scratch_types=[
        pltpu.VMEM((IDXCH, 128), jnp.int32),
        pltpu.VMEM((IDXCH, 128), jnp.int32),
        pltpu.VMEM((BPW,), jnp.float32),
        pltpu.SemaphoreType.DMA,
        pltpu.SemaphoreType.DMA,
        pltpu.SemaphoreType.DMA,
        pltpu.SemaphoreType.DMA,
    ] + _WIN_SCRATCH,
)
def _sc_dot_kernel(ti_hbm, cj_hbm, emb_hbm, out_hbm,
                   idx_t, idx_c, outv, s_t0, s_c0, s_t1, s_c1, *wins):
    wid = lax.axis_index("s") * NC + lax.axis_index("c")
    base = wid * BPW
    row0 = wid * IDXCH

    pltpu.sync_copy(ti_hbm.at[pl.ds(row0, IDXCH)], idx_t)
    pltpu.sync_copy(cj_hbm.at[pl.ds(row0, IDXCH)], idx_c)

    lanes = lax.iota(jnp.int32, L)
    sems = ((s_t0, s_c0), (s_t1, s_c1))

    def wbuf(bank, role, k):
        return wins[bank * 2 * L + role * L + k]

    def load_ids(g):
        j = g // (128 // L)
        r = (g % (128 // L)) * L
        return idx_t[j, pl.ds(r, L)], idx_c[j, pl.ds(r, L)]

    def issue(g, bank):
        ivt, ivc = load_ids(g)
        st, sc_ = sems[bank]
        for k in range(L):
            rt = pl.multiple_of(ivt[k] & -8, 8)
            rc = pl.multiple_of(ivc[k] & -8, 8)
            pltpu.async_copy(emb_hbm.at[pl.ds(rt, 8), :], wbuf(bank, 0, k), st)
            pltpu.async_copy(emb_hbm.at[pl.ds(rc, 8), :], wbuf(bank, 1, k), sc_)

    def drain(bank):
        st, sc_ = sems[bank]
        dummy = emb_hbm.at[pl.ds(0, 8), :]
        for k in range(L):
            pltpu.make_async_copy(dummy, wbuf(bank, 0, k), st).wait()
            pltpu.make_async_copy(dummy, wbuf(bank, 1, k), sc_).wait()

    def compute(g, bank):
        ivt, ivc = load_ids(g)
        acc = jnp.zeros((L,), jnp.float32)
        for k in range(L):
            rowt = jnp.full((L,), ivt[k] & 7, jnp.int32)
            rowc = jnp.full((L,), ivc[k] & 7, jnp.int32)
            s = jnp.zeros((L,), jnp.float32)
            for q in range(D // L):
                cols = q * L + lanes
                tv = plsc.load_gather(wbuf(bank, 0, k), [rowt, cols])
                cv = plsc.load_gather(wbuf(bank, 1, k), [rowc, cols])
                s = s + tv * cv
            acc = jnp.where(lanes == k, jnp.sum(s), acc)
        outv[pl.ds(g * L, L)] = 1.0 / (1.0 + jnp.exp(-acc))

    issue(0, 0)
    issue(1, 1)

    def body(h, _):
        g = 2 * h
        drain(0)
        compute(g, 0)
        issue(g + 2, 0)
        drain(1)
        compute(g + 1, 1)
        issue(g + 3, 1)
        return 0

    lax.fori_loop(0, NG // 2 - 1, body, 0)
    drain(0)
    compute(NG - 2, 0)
    drain(1)
    compute(NG - 1, 1)

    pltpu.sync_copy(outv, out_hbm.at[pl.ds(base, BPW)])


def kernel(target_i, context_j, label, emb):
    ti = target_i.reshape(B // 128, 128)
    cj = context_j.reshape(B // 128, 128)
    out = _sc_dot_kernel(ti, cj, emb)
    return (out, label.astype(jnp.float32))
